# baseline (speedup 1.0000x reference)
import dataclasses
import functools

import jax
import jax.numpy as jnp
from jax import lax
from jax.experimental import pallas as pl
from jax.experimental.pallas import tpu as pltpu
from jax.experimental.pallas import tpu_sc as plsc

N = 10000
E = 320000
D = 128
H = 128

NPAD = 10240
NC = 2
NS = 16
NW = NC * NS
BPW = NPAD // NW
EPW = E // NW
CH = 80
NCHUNK = EPW // CH

SCAN = 512
CHV = 2048
CAPT = (E // CHV + 2) * CHV
GCH = 64

_mesh = plsc.VectorSubcoreMesh(core_axis_name="c", subcore_axis_name="s")
_cp = pltpu.CompilerParams()
if "needs_layout_passes" in pltpu.CompilerParams.__dataclass_fields__:
    _cp = dataclasses.replace(_cp, needs_layout_passes=False)


@functools.partial(
    pl.kernel,
    mesh=_mesh,
    out_type=jax.ShapeDtypeStruct((NC, NPAD, 128), jnp.float32),
    scratch_types=[
        pltpu.VMEM((1, CH), jnp.int32),
        pltpu.VMEM((1, CH), jnp.int32),
        pltpu.VMEM((CH, 128), jnp.float32),
        pltpu.VMEM((80, 128), jnp.float32),
        pltpu.VMEM_SHARED((NPAD, 128), jnp.float32),
        pltpu.SemaphoreType.DMA,
    ],
)
def _mean_deg_sc(h_hbm, src_hbm, dst_hbm, sums_hbm,
                 idx_src, idx_dst, rows, zbuf, acc_sh, sem):
    c = lax.axis_index("c")
    s = lax.axis_index("s")
    wid = c * NS + s

    zero16 = jnp.zeros((16,), jnp.float32)

    @pl.loop(0, 80)
    def _(i):
        for j in range(8):
            zbuf[i, pl.ds(j * 16, 16)] = zero16

    rows_per_tile = NPAD // NS
    base_row = s * rows_per_tile

    @pl.loop(0, rows_per_tile // 80)
    def _(k):
        pltpu.sync_copy(zbuf, acc_sh.at[pl.ds(base_row + k * 80, 80)])

    plsc.subcore_barrier()

    ebase = wid * EPW

    @pl.loop(0, NCHUNK)
    def _(i):
        off = ebase + i * CH
        pltpu.sync_copy(src_hbm.at[pl.ds(off, CH)], idx_src.at[0])
        pltpu.sync_copy(dst_hbm.at[pl.ds(off, CH)], idx_dst.at[0])
        pltpu.async_copy(h_hbm.at[idx_src.at[0]], rows, sem).wait()
        pltpu.sync_copy(rows, acc_sh.at[idx_dst.at[0]], add=True)

    plsc.subcore_barrier()

    @pl.loop(0, rows_per_tile // 80)
    def _(k):
        r = base_row + k * 80
        pltpu.sync_copy(acc_sh.at[pl.ds(r, 80)], sums_hbm.at[c].at[pl.ds(r, 80)])


@functools.partial(
    pl.kernel,
    mesh=_mesh,
    out_type=[
        jax.ShapeDtypeStruct((NW, CAPT), jnp.int32),
        jax.ShapeDtypeStruct((NW, CAPT), jnp.int32),
        jax.ShapeDtypeStruct((NW * 16,), jnp.int32),
    ],
    scratch_types=[
        pltpu.VMEM((1, SCAN), jnp.int32),
        pltpu.VMEM((1, SCAN), jnp.int32),
        pltpu.VMEM((CHV + 16,), jnp.int32),
        pltpu.VMEM((CHV + 16,), jnp.int32),
        pltpu.VMEM((16,), jnp.int32),
    ],
    compiler_params=_cp,
)
def _bin_edges_sc(src_hbm, dst_hbm, lsrc_hbm, ldst_hbm, cnt_hbm,
                  in_src, in_dst, buf_src, buf_dst, cstage):
    c = lax.axis_index("c")
    s = lax.axis_index("s")
    wid = c * NS + s
    lo = wid * BPW
    hi = lo + BPW
    src_sent = wid * 311 % N
    dst_sent = lo + BPW

    def flush(carry):
        cnt, nf = carry
        pltpu.sync_copy(buf_src.at[pl.ds(0, CHV)],
                        lsrc_hbm.at[wid].at[pl.ds(nf * CHV, CHV)])
        pltpu.sync_copy(buf_dst.at[pl.ds(0, CHV)],
                        ldst_hbm.at[wid].at[pl.ds(nf * CHV, CHV)])
        ts = buf_src[pl.ds(CHV, 16)]
        td = buf_dst[pl.ds(CHV, 16)]
        buf_src[pl.ds(0, 16)] = ts
        buf_dst[pl.ds(0, 16)] = td
        return (cnt - CHV, nf + 1)

    def scan_step(i, carry):
        pltpu.sync_copy(src_hbm.at[pl.ds(i * SCAN, SCAN)], in_src.at[0])
        pltpu.sync_copy(dst_hbm.at[pl.ds(i * SCAN, SCAN)], in_dst.at[0])

        def sub(kk, cc):
            cnt, nf = cc
            dv = in_dst[0, pl.ds(kk * 16, 16)]
            sv = in_src[0, pl.ds(kk * 16, 16)]
            m = (dv >= lo) & (dv < hi)
            plsc.store_compressed(buf_src.at[pl.ds(cnt, 16)], sv, mask=m)
            plsc.store_compressed(buf_dst.at[pl.ds(cnt, 16)], dv, mask=m)
            cnt = cnt + plsc.all_reduce_population_count(m)[0]
            return lax.cond(cnt >= CHV, flush, lambda cc2: cc2, (cnt, nf))

        return lax.fori_loop(0, SCAN // 16, sub, carry)

    cnt, nf = lax.fori_loop(0, E // SCAN, scan_step, (jnp.int32(0), jnp.int32(0)))

    sent_s = jnp.full((16,), src_sent, jnp.int32)
    sent_d = jnp.full((16,), dst_sent, jnp.int32)
    buf_src[pl.ds(cnt, 16)] = sent_s
    buf_dst[pl.ds(cnt, 16)] = sent_d
    cnt = cnt + ((16 - (cnt & 15)) & 15)

    def pad16(j, cc):
        cnt2 = cc

        @pl.when((cnt2 & (GCH - 1)) != 0)
        def _():
            buf_src[pl.ds(cnt2, 16)] = sent_s
            buf_dst[pl.ds(cnt2, 16)] = sent_d

        return lax.cond((cnt2 & (GCH - 1)) != 0, lambda v: v + 16, lambda v: v, cnt2)

    cnt = lax.fori_loop(0, GCH // 16 - 1, pad16, cnt)

    pltpu.sync_copy(buf_src.at[pl.ds(0, CHV)],
                    lsrc_hbm.at[wid].at[pl.ds(nf * CHV, CHV)])
    pltpu.sync_copy(buf_dst.at[pl.ds(0, CHV)],
                    ldst_hbm.at[wid].at[pl.ds(nf * CHV, CHV)])

    total = nf * CHV + cnt
    cstage[pl.ds(0, 16)] = jnp.full((16,), total, jnp.int32)
    pltpu.sync_copy(cstage.at[pl.ds(0, 16)], cnt_hbm.at[pl.ds(wid * 16, 16)])


@functools.partial(
    pl.kernel,
    mesh=_mesh,
    out_type=jax.ShapeDtypeStruct((NPAD, 128), jnp.float32),
    scratch_types=[
        pltpu.VMEM((1, GCH), jnp.int32),
        pltpu.VMEM((1, GCH + 16), jnp.int32),
        pltpu.VMEM((GCH, 128), jnp.float32),
        pltpu.VMEM((BPW + 16, 128), jnp.float32),
        pltpu.VMEM((1, 16), jnp.int32),
        pltpu.SemaphoreType.DMA,
    ],
    compiler_params=_cp,
)
def _max_sc(h_hbm, lsrc_hbm, ldst_hbm, cnt_hbm, out_hbm,
            isrc, idst, rows, acc, cvec, sem):
    c = lax.axis_index("c")
    s = lax.axis_index("s")
    wid = c * NS + s
    nbase = wid * BPW

    neg = jnp.full((16,), -3.4e38, jnp.float32)

    @pl.loop(0, BPW + 16)
    def _(i):
        for j in range(8):
            acc[i, pl.ds(j * 16, 16)] = neg

    pltpu.sync_copy(cnt_hbm.at[pl.ds(wid * 16, 16)], cvec.at[0])
    n = lax.reduce_max(cvec[0, :], (0,))
    nch = lax.shift_right_logical(n, 6)

    def chunk(g, carry):
        off = g * GCH
        pltpu.sync_copy(lsrc_hbm.at[wid].at[pl.ds(off, GCH)], isrc.at[0])
        pltpu.sync_copy(ldst_hbm.at[wid].at[pl.ds(off, GCH)],
                        idst.at[0, pl.ds(0, GCH)])
        pltpu.async_copy(h_hbm.at[isrc.at[0]], rows, sem).wait()

        @pl.loop(0, GCH)
        def _(e):
            dl = idst[0, pl.ds(e, 16)][0] - nbase
            for j in range(8):
                sl = pl.ds(j * 16, 16)
                acc[dl, sl] = jnp.maximum(acc[dl, sl], rows[e, sl])

        return carry

    lax.fori_loop(0, nch, chunk, jnp.int32(0))

    @pl.loop(0, BPW // GCH)
    def _(k):
        r = k * GCH
        pltpu.sync_copy(acc.at[pl.ds(r, GCH)], out_hbm.at[pl.ds(nbase + r, GCH)])


def _max_agg_sc(h, lsrc, ldst, cnts, deg):
    mx = _max_sc(h, lsrc, ldst, cnts)
    return jnp.where((deg > 0)[:, None], mx[:N], h)


def _mean_agg_sc(h, src, dst, deg):
    sums2 = _mean_deg_sc(h, src, dst)
    sums = sums2[0, :N] + sums2[1, :N]
    mean = sums / jnp.maximum(deg, 1.0)[:, None]
    return jnp.where((deg > 0)[:, None], mean, h)


def _dense_bn_body(h_ref, w_ref, b_ref, g_ref, bb_ref, o_ref):
    h = h_ref[...]
    a = jnp.maximum(
        jnp.dot(h, w_ref[...], preferred_element_type=jnp.float32) + b_ref[...],
        0.0,
    )
    mu = jnp.mean(a, axis=0, keepdims=True)
    var = jnp.mean((a - mu) ** 2, axis=0, keepdims=True)
    o_ref[...] = (a - mu) * lax.rsqrt(var + 1e-5) * g_ref[...] + bb_ref[...]


def _dense_bn(h, W, b, g, bb):
    return pl.pallas_call(
        _dense_bn_body,
        out_shape=jax.ShapeDtypeStruct((N, H), jnp.float32),
    )(h, W, b.reshape(1, H), g.reshape(1, H), bb.reshape(1, H))


def _head_body(h_ref, wd_ref, bd_ref, gf_ref, bf_ref, wp_ref, bp_ref, o_ref):
    h = h_ref[...]
    a = jnp.maximum(
        jnp.dot(h, wd_ref[...], preferred_element_type=jnp.float32) + bd_ref[...],
        0.0,
    )
    mu = jnp.mean(a, axis=0, keepdims=True)
    var = jnp.mean((a - mu) ** 2, axis=0, keepdims=True)
    hb = (a - mu) * lax.rsqrt(var + 1e-5) * gf_ref[...] + bf_ref[...]
    hg = jnp.tanh(jnp.mean(hb, axis=0, keepdims=True))
    o_ref[...] = jnp.dot(hg, wp_ref[...], preferred_element_type=jnp.float32) + bp_ref[...]


def kernel(x, edge_index, W1, b1, g1, bb1, W2, b2, g2, bb2, Wd, bd, gf, bf, Wp, bp):
    src = edge_index[0]
    dst = edge_index[1]

    deg = jax.ops.segment_sum(jnp.ones((E,), jnp.float32), dst, num_segments=N)
    lsrc, ldst, cnts = _bin_edges_sc(src, dst)
    h = _mean_agg_sc(x, src, dst, deg)
    h = _dense_bn(h, W1, b1, g1, bb1)
    h = _max_agg_sc(h, lsrc, ldst, cnts, deg)
    h = _mean_agg_sc(h, src, dst, deg)
    h = _dense_bn(h, W2, b2, g2, bb2)
    h = _max_agg_sc(h, lsrc, ldst, cnts, deg)

    out = pl.pallas_call(
        _head_body,
        out_shape=jax.ShapeDtypeStruct((1, 1), jnp.float32),
    )(h, Wd, bd.reshape(1, H), gf.reshape(1, H), bf.reshape(1, H), Wp, bp.reshape(1, 1))
    return out


# speedup vs baseline: 1.2876x; 1.2876x over previous
import dataclasses
import functools

import jax
import jax.numpy as jnp
from jax import lax
from jax.experimental import pallas as pl
from jax.experimental.pallas import tpu as pltpu
from jax.experimental.pallas import tpu_sc as plsc

N = 10000
E = 320000
D = 128
H = 128

NPAD = 10240
NC = 2
NS = 16
NW = NC * NS
BPW = NPAD // NW

SCAN = 4000
NSTEP = E // SCAN
CHV = 2048
CAPT = (E // CHV + 2) * CHV
GCH = 128
PAIR = 2 * GCH
LBS = 2048

_mesh = plsc.VectorSubcoreMesh(core_axis_name="c", subcore_axis_name="s")
_cp = pltpu.CompilerParams()
if "needs_layout_passes" in pltpu.CompilerParams.__dataclass_fields__:
    _cp = dataclasses.replace(_cp, needs_layout_passes=False)


@functools.partial(
    pl.kernel,
    mesh=_mesh,
    out_type=[
        jax.ShapeDtypeStruct((NW, CAPT), jnp.int32),
        jax.ShapeDtypeStruct((NW, CAPT), jnp.int32),
        jax.ShapeDtypeStruct((NW * 16,), jnp.int32),
    ],
    scratch_types=[
        pltpu.VMEM((SCAN,), jnp.int32),
        pltpu.VMEM((SCAN,), jnp.int32),
        pltpu.VMEM((SCAN,), jnp.int32),
        pltpu.VMEM((SCAN,), jnp.int32),
        pltpu.VMEM((CHV + 16,), jnp.int32),
        pltpu.VMEM((CHV + 16,), jnp.int32),
        pltpu.VMEM((16,), jnp.int32),
        pltpu.SemaphoreType.DMA,
        pltpu.SemaphoreType.DMA,
    ],
    compiler_params=_cp,
)
def _bin_edges_sc(src_hbm, dst_hbm, lsrc_hbm, ldst_hbm, cnt_hbm,
                  in_src0, in_src1, in_dst0, in_dst1,
                  buf_src, buf_dst, cstage, sem0, sem1):
    c = lax.axis_index("c")
    s = lax.axis_index("s")
    wid = c * NS + s
    lo = wid * BPW
    hi = lo + BPW
    src_sent = wid * 311 % N
    dst_sent = lo + BPW
    sems = (sem0, sem1)
    srcb = (in_src0, in_src1)
    dstb = (in_dst0, in_dst1)

    def issue(b, i):
        off = i * SCAN
        pltpu.async_copy(src_hbm.at[pl.ds(off, SCAN)], srcb[b], sems[b])
        pltpu.async_copy(dst_hbm.at[pl.ds(off, SCAN)], dstb[b], sems[b])

    def wait(b, i):
        off = i * SCAN
        pltpu.make_async_copy(src_hbm.at[pl.ds(off, SCAN)], srcb[b], sems[b]).wait()
        pltpu.make_async_copy(dst_hbm.at[pl.ds(off, SCAN)], dstb[b], sems[b]).wait()

    def flush(carry):
        cnt, nf = carry
        pltpu.sync_copy(buf_src.at[pl.ds(0, CHV)],
                        lsrc_hbm.at[wid].at[pl.ds(nf * CHV, CHV)])
        pltpu.sync_copy(buf_dst.at[pl.ds(0, CHV)],
                        ldst_hbm.at[wid].at[pl.ds(nf * CHV, CHV)])
        ts = buf_src[pl.ds(CHV, 16)]
        td = buf_dst[pl.ds(CHV, 16)]
        buf_src[pl.ds(0, 16)] = ts
        buf_dst[pl.ds(0, 16)] = td
        return (cnt - CHV, nf + 1)

    def process(b, carry):
        def sub(kk, cc):
            cnt, nf = cc
            dv = dstb[b][pl.ds(kk * 16, 16)]
            sv = srcb[b][pl.ds(kk * 16, 16)]
            m = (dv >= lo) & (dv < hi)
            plsc.store_compressed(buf_src.at[pl.ds(cnt, 16)], sv, mask=m)
            plsc.store_compressed(buf_dst.at[pl.ds(cnt, 16)], dv, mask=m)
            cnt = cnt + plsc.all_reduce_population_count(m)[0]
            return lax.cond(cnt >= CHV, flush, lambda cc2: cc2, (cnt, nf))

        return lax.fori_loop(0, SCAN // 16, sub, carry)

    issue(0, 0)

    def pair_step(p, carry):
        issue(1, 2 * p + 1)
        wait(0, 2 * p)
        carry = process(0, carry)

        @pl.when(2 * p + 2 < NSTEP)
        def _():
            issue(0, 2 * p + 2)

        wait(1, 2 * p + 1)
        return process(1, carry)

    cnt, nf = lax.fori_loop(0, NSTEP // 2, pair_step,
                            (jnp.int32(0), jnp.int32(0)))

    sent_s = jnp.full((16,), src_sent, jnp.int32)
    sent_d = jnp.full((16,), dst_sent, jnp.int32)
    buf_src[pl.ds(cnt, 16)] = sent_s
    buf_dst[pl.ds(cnt, 16)] = sent_d
    cnt = cnt + ((16 - (cnt & 15)) & 15)

    def pad16(j, cnt2):
        @pl.when((cnt2 & (PAIR - 1)) != 0)
        def _():
            buf_src[pl.ds(cnt2, 16)] = sent_s
            buf_dst[pl.ds(cnt2, 16)] = sent_d

        return lax.cond((cnt2 & (PAIR - 1)) != 0,
                        lambda v: v + 16, lambda v: v, cnt2)

    cnt = lax.fori_loop(0, PAIR // 16 - 1, pad16, cnt)

    pltpu.sync_copy(buf_src.at[pl.ds(0, CHV)],
                    lsrc_hbm.at[wid].at[pl.ds(nf * CHV, CHV)])
    pltpu.sync_copy(buf_dst.at[pl.ds(0, CHV)],
                    ldst_hbm.at[wid].at[pl.ds(nf * CHV, CHV)])

    total = nf * CHV + cnt
    cstage[pl.ds(0, 16)] = jnp.full((16,), total, jnp.int32)
    pltpu.sync_copy(cstage.at[pl.ds(0, 16)], cnt_hbm.at[pl.ds(wid * 16, 16)])


def _make_agg(op):
    init = 0.0 if op == "add" else -3.4e38

    @functools.partial(
        pl.kernel,
        mesh=_mesh,
        out_type=jax.ShapeDtypeStruct((NPAD, 128), jnp.float32),
        scratch_types=[
            pltpu.VMEM((LBS,), jnp.int32),
            pltpu.VMEM((LBS + 16,), jnp.int32),
            pltpu.VMEM((2, GCH, 128), jnp.float32),
            pltpu.VMEM((BPW + 16, 128), jnp.float32),
            pltpu.VMEM((1, 16), jnp.int32),
            pltpu.SemaphoreType.DMA,
            pltpu.SemaphoreType.DMA,
        ],
        compiler_params=_cp,
    )
    def _agg(h_hbm, lsrc_hbm, ldst_hbm, cnt_hbm, out_hbm,
             bsrc, bdst, rows2, acc, cvec, sem0, sem1):
        c = lax.axis_index("c")
        s = lax.axis_index("s")
        wid = c * NS + s
        nbase = wid * BPW
        sems = (sem0, sem1)

        ival = jnp.full((16,), init, jnp.float32)

        @pl.loop(0, BPW + 16)
        def _(i):
            for j in range(8):
                acc[i, pl.ds(j * 16, 16)] = ival

        pltpu.sync_copy(cnt_hbm.at[pl.ds(wid * 16, 16)], cvec.at[0])
        n = lax.reduce_max(cvec[0, :], (0,))
        nblk = lax.shift_right_logical(n + LBS - 1, 11)

        def gissue(gb, ck):
            pltpu.async_copy(
                h_hbm.at[bsrc.at[pl.ds(ck * GCH, GCH)]], rows2.at[gb], sems[gb])

        def gwait(gb, ck):
            pltpu.make_async_copy(
                h_hbm.at[bsrc.at[pl.ds(ck * GCH, GCH)]], rows2.at[gb],
                sems[gb]).wait()

        def compute(gb, ck):
            @pl.loop(0, GCH)
            def _(e):
                dl = bdst[pl.ds(ck * GCH + e, 16)][0] - nbase
                for j in range(8):
                    sl = pl.ds(j * 16, 16)
                    if op == "add":
                        acc[dl, sl] = acc[dl, sl] + rows2[gb, e, sl]
                    else:
                        acc[dl, sl] = jnp.maximum(acc[dl, sl], rows2[gb, e, sl])

        def block(b, carry):
            off = b * LBS
            pltpu.sync_copy(lsrc_hbm.at[wid].at[pl.ds(off, LBS)], bsrc)
            pltpu.sync_copy(ldst_hbm.at[wid].at[pl.ds(off, LBS)],
                            bdst.at[pl.ds(0, LBS)])
            rem = n - off
            pr = jnp.minimum(jnp.int32(LBS // PAIR),
                             lax.shift_right_logical(rem, 8))
            gissue(0, 0)

            def pair(p, cc):
                gissue(1, 2 * p + 1)
                gwait(0, 2 * p)
                compute(0, 2 * p)

                @pl.when(p + 1 < pr)
                def _():
                    gissue(0, 2 * p + 2)

                gwait(1, 2 * p + 1)
                compute(1, 2 * p + 1)
                return cc

            return lax.fori_loop(0, pr, pair, carry)

        lax.fori_loop(0, nblk, block, jnp.int32(0))

        @pl.loop(0, BPW // 64)
        def _(k):
            r = k * 64
            pltpu.sync_copy(acc.at[pl.ds(r, 64)],
                            out_hbm.at[pl.ds(nbase + r, 64)])

    return _agg


_agg_add = _make_agg("add")
_agg_max = _make_agg("max")


def _mean_agg(h, lsrc, ldst, cnts, deg):
    sums = _agg_add(h, lsrc, ldst, cnts)
    mean = sums[:N] / jnp.maximum(deg, 1.0)[:, None]
    return jnp.where((deg > 0)[:, None], mean, h)


def _max_agg(h, lsrc, ldst, cnts, deg):
    mx = _agg_max(h, lsrc, ldst, cnts)
    return jnp.where((deg > 0)[:, None], mx[:N], h)


def _dense_bn_body(h_ref, w_ref, b_ref, g_ref, bb_ref, o_ref):
    h = h_ref[...]
    a = jnp.maximum(
        jnp.dot(h, w_ref[...], preferred_element_type=jnp.float32) + b_ref[...],
        0.0,
    )
    mu = jnp.mean(a, axis=0, keepdims=True)
    var = jnp.mean((a - mu) ** 2, axis=0, keepdims=True)
    o_ref[...] = (a - mu) * lax.rsqrt(var + 1e-5) * g_ref[...] + bb_ref[...]


def _dense_bn(h, W, b, g, bb):
    return pl.pallas_call(
        _dense_bn_body,
        out_shape=jax.ShapeDtypeStruct((N, H), jnp.float32),
    )(h, W, b.reshape(1, H), g.reshape(1, H), bb.reshape(1, H))


def _head_body(h_ref, wd_ref, bd_ref, gf_ref, bf_ref, wp_ref, bp_ref, o_ref):
    h = h_ref[...]
    a = jnp.maximum(
        jnp.dot(h, wd_ref[...], preferred_element_type=jnp.float32) + bd_ref[...],
        0.0,
    )
    mu = jnp.mean(a, axis=0, keepdims=True)
    var = jnp.mean((a - mu) ** 2, axis=0, keepdims=True)
    hb = (a - mu) * lax.rsqrt(var + 1e-5) * gf_ref[...] + bf_ref[...]
    hg = jnp.tanh(jnp.mean(hb, axis=0, keepdims=True))
    o_ref[...] = jnp.dot(hg, wp_ref[...], preferred_element_type=jnp.float32) + bp_ref[...]


def kernel(x, edge_index, W1, b1, g1, bb1, W2, b2, g2, bb2, Wd, bd, gf, bf, Wp, bp):
    src = edge_index[0]
    dst = edge_index[1]

    deg = jax.ops.segment_sum(jnp.ones((E,), jnp.float32), dst, num_segments=N)
    lsrc, ldst, cnts = _bin_edges_sc(src, dst)
    h = _mean_agg(x, lsrc, ldst, cnts, deg)
    h = _dense_bn(h, W1, b1, g1, bb1)
    h = _max_agg(h, lsrc, ldst, cnts, deg)
    h = _mean_agg(h, lsrc, ldst, cnts, deg)
    h = _dense_bn(h, W2, b2, g2, bb2)
    h = _max_agg(h, lsrc, ldst, cnts, deg)

    out = pl.pallas_call(
        _head_body,
        out_shape=jax.ShapeDtypeStruct((1, 1), jnp.float32),
    )(h, Wd, bd.reshape(1, H), gf.reshape(1, H), bf.reshape(1, H), Wp, bp.reshape(1, 1))
    return out


# speedup vs baseline: 1.8402x; 1.4292x over previous
import dataclasses
import functools

import jax
import jax.numpy as jnp
from jax import lax
from jax.experimental import pallas as pl
from jax.experimental.pallas import tpu as pltpu
from jax.experimental.pallas import tpu_sc as plsc

N = 10000
E = 320000
D = 128
H = 128

NPAD = 10240
NC = 2
NS = 16
NW = NC * NS
BPW = NPAD // NW

SCAN = 4000
NSTEP = E // SCAN
CHV = 2048
CAPT = (E // CHV + 2) * CHV
GCH = 128
PAIR = 2 * GCH
LBS = 2048

_mesh = plsc.VectorSubcoreMesh(core_axis_name="c", subcore_axis_name="s")
_cp = pltpu.CompilerParams()
if "needs_layout_passes" in pltpu.CompilerParams.__dataclass_fields__:
    _cp = dataclasses.replace(_cp, needs_layout_passes=False)


@functools.partial(
    pl.kernel,
    mesh=_mesh,
    out_type=[
        jax.ShapeDtypeStruct((NW, CAPT), jnp.int32),
        jax.ShapeDtypeStruct((NW, CAPT), jnp.int32),
        jax.ShapeDtypeStruct((NW * 16,), jnp.int32),
    ],
    scratch_types=[
        pltpu.VMEM((SCAN,), jnp.int32),
        pltpu.VMEM((SCAN,), jnp.int32),
        pltpu.VMEM((SCAN,), jnp.int32),
        pltpu.VMEM((SCAN,), jnp.int32),
        pltpu.VMEM((CHV + 16,), jnp.int32),
        pltpu.VMEM((CHV + 16,), jnp.int32),
        pltpu.VMEM((16,), jnp.int32),
        pltpu.SemaphoreType.DMA,
        pltpu.SemaphoreType.DMA,
    ],
    compiler_params=_cp,
)
def _bin_edges_sc(src_hbm, dst_hbm, lsrc_hbm, ldst_hbm, cnt_hbm,
                  in_src0, in_src1, in_dst0, in_dst1,
                  buf_src, buf_dst, cstage, sem0, sem1):
    c = lax.axis_index("c")
    s = lax.axis_index("s")
    wid = c * NS + s
    lo = wid * BPW
    hi = lo + BPW
    src_sent = wid * 311 % N
    dst_sent = lo + BPW
    sems = (sem0, sem1)
    srcb = (in_src0, in_src1)
    dstb = (in_dst0, in_dst1)

    def issue(b, i):
        off = i * SCAN
        pltpu.async_copy(src_hbm.at[pl.ds(off, SCAN)], srcb[b], sems[b])
        pltpu.async_copy(dst_hbm.at[pl.ds(off, SCAN)], dstb[b], sems[b])

    def wait(b, i):
        off = i * SCAN
        pltpu.make_async_copy(src_hbm.at[pl.ds(off, SCAN)], srcb[b], sems[b]).wait()
        pltpu.make_async_copy(dst_hbm.at[pl.ds(off, SCAN)], dstb[b], sems[b]).wait()

    def flush(carry):
        cnt, nf = carry
        pltpu.sync_copy(buf_src.at[pl.ds(0, CHV)],
                        lsrc_hbm.at[wid].at[pl.ds(nf * CHV, CHV)])
        pltpu.sync_copy(buf_dst.at[pl.ds(0, CHV)],
                        ldst_hbm.at[wid].at[pl.ds(nf * CHV, CHV)])
        ts = buf_src[pl.ds(CHV, 16)]
        td = buf_dst[pl.ds(CHV, 16)]
        buf_src[pl.ds(0, 16)] = ts
        buf_dst[pl.ds(0, 16)] = td
        return (cnt - CHV, nf + 1)

    def process(b, carry):
        def sub(kk, cc):
            cnt, nf = cc
            dv = dstb[b][pl.ds(kk * 16, 16)]
            sv = srcb[b][pl.ds(kk * 16, 16)]
            m = (dv >= lo) & (dv < hi)
            plsc.store_compressed(buf_src.at[pl.ds(cnt, 16)], sv, mask=m)
            plsc.store_compressed(buf_dst.at[pl.ds(cnt, 16)], dv, mask=m)
            cnt = cnt + plsc.all_reduce_population_count(m)[0]
            return lax.cond(cnt >= CHV, flush, lambda cc2: cc2, (cnt, nf))

        return lax.fori_loop(0, SCAN // 16, sub, carry)

    issue(0, 0)

    def pair_step(p, carry):
        issue(1, 2 * p + 1)
        wait(0, 2 * p)
        carry = process(0, carry)

        @pl.when(2 * p + 2 < NSTEP)
        def _():
            issue(0, 2 * p + 2)

        wait(1, 2 * p + 1)
        return process(1, carry)

    cnt, nf = lax.fori_loop(0, NSTEP // 2, pair_step,
                            (jnp.int32(0), jnp.int32(0)))

    sent_s = jnp.full((16,), src_sent, jnp.int32)
    sent_d = jnp.full((16,), dst_sent, jnp.int32)
    buf_src[pl.ds(cnt, 16)] = sent_s
    buf_dst[pl.ds(cnt, 16)] = sent_d
    cnt = cnt + ((16 - (cnt & 15)) & 15)

    def pad16(j, cnt2):
        @pl.when((cnt2 & (PAIR - 1)) != 0)
        def _():
            buf_src[pl.ds(cnt2, 16)] = sent_s
            buf_dst[pl.ds(cnt2, 16)] = sent_d

        return lax.cond((cnt2 & (PAIR - 1)) != 0,
                        lambda v: v + 16, lambda v: v, cnt2)

    cnt = lax.fori_loop(0, PAIR // 16 - 1, pad16, cnt)

    pltpu.sync_copy(buf_src.at[pl.ds(0, CHV)],
                    lsrc_hbm.at[wid].at[pl.ds(nf * CHV, CHV)])
    pltpu.sync_copy(buf_dst.at[pl.ds(0, CHV)],
                    ldst_hbm.at[wid].at[pl.ds(nf * CHV, CHV)])

    total = nf * CHV + cnt
    cstage[pl.ds(0, 16)] = jnp.full((16,), total, jnp.int32)
    pltpu.sync_copy(cstage.at[pl.ds(0, 16)], cnt_hbm.at[pl.ds(wid * 16, 16)])


NHALF = NPAD // 2


@functools.partial(
    pl.kernel,
    mesh=_mesh,
    out_type=jax.ShapeDtypeStruct((NPAD, 128), jnp.float32),
    scratch_types=[
        pltpu.VMEM((LBS,), jnp.int32),
        pltpu.VMEM((LBS,), jnp.int32),
        pltpu.VMEM((2, GCH, 128), jnp.float32),
        pltpu.VMEM((2, 128), jnp.int32),
        pltpu.VMEM((107, 128), jnp.float32),
        pltpu.VMEM((1, 16), jnp.int32),
        pltpu.VMEM_SHARED((NHALF + 16, 128), jnp.float32),
        pltpu.SemaphoreType.DMA,
        pltpu.SemaphoreType.DMA,
    ],
    compiler_params=_cp,
)
def _sum_sc(h_hbm, lsrc_hbm, ldst_hbm, cnt_hbm, out_hbm,
            bsrc, bdst, rows2, ilocb, zbuf, cvec, acc_sh, sem0, sem1):
    c = lax.axis_index("c")
    s = lax.axis_index("s")
    wid = c * NS + s
    sc_base = c * NHALF
    sent_val = (wid + 1) * BPW
    sems = (sem0, sem1)

    zero16 = jnp.zeros((16,), jnp.float32)

    @pl.loop(0, 107)
    def _(i):
        for j in range(8):
            zbuf[i, pl.ds(j * 16, 16)] = zero16

    @pl.loop(0, 3)
    def _(k):
        pltpu.sync_copy(zbuf.at[pl.ds(0, 107)],
                        acc_sh.at[pl.ds(s * 321 + k * 107, 107)])

    plsc.subcore_barrier()

    pltpu.sync_copy(cnt_hbm.at[pl.ds(wid * 16, 16)], cvec.at[0])
    n = lax.reduce_max(cvec[0, :], (0,))
    nblk = lax.shift_right_logical(n + LBS - 1, 11)

    def gissue(gb, ck):
        pltpu.async_copy(
            h_hbm.at[bsrc.at[pl.ds(ck * GCH, GCH)]], rows2.at[gb], sems[gb])

    def gwait(gb, ck):
        pltpu.make_async_copy(
            h_hbm.at[bsrc.at[pl.ds(ck * GCH, GCH)]], rows2.at[gb],
            sems[gb]).wait()

    def scatter(gb, ck):
        for j in range(8):
            dv = bdst[pl.ds(ck * GCH + j * 16, 16)]
            loc = jnp.where(dv == sent_val, jnp.int32(NHALF), dv - sc_base)
            ilocb[gb, pl.ds(j * 16, 16)] = loc
        pltpu.sync_copy(rows2.at[gb], acc_sh.at[ilocb.at[gb]], add=True)

    def block(b, carry):
        off = b * LBS
        pltpu.sync_copy(lsrc_hbm.at[wid].at[pl.ds(off, LBS)], bsrc)
        pltpu.sync_copy(ldst_hbm.at[wid].at[pl.ds(off, LBS)], bdst)
        rem = n - off
        pr = jnp.minimum(jnp.int32(LBS // PAIR),
                         lax.shift_right_logical(rem, 8))
        gissue(0, 0)

        def pair(p, cc):
            gissue(1, 2 * p + 1)
            gwait(0, 2 * p)
            scatter(0, 2 * p)

            @pl.when(p + 1 < pr)
            def _():
                gissue(0, 2 * p + 2)

            gwait(1, 2 * p + 1)
            scatter(1, 2 * p + 1)
            return cc

        return lax.fori_loop(0, pr, pair, carry)

    lax.fori_loop(0, nblk, block, jnp.int32(0))

    @pl.loop(0, BPW // 64)
    def _(k):
        r = s * BPW + k * 64
        pltpu.sync_copy(acc_sh.at[pl.ds(r, 64)],
                        out_hbm.at[pl.ds(sc_base + r, 64)])


def _make_agg(op):
    init = 0.0 if op == "add" else -3.4e38

    @functools.partial(
        pl.kernel,
        mesh=_mesh,
        out_type=jax.ShapeDtypeStruct((NPAD, 128), jnp.float32),
        scratch_types=[
            pltpu.VMEM((LBS,), jnp.int32),
            pltpu.VMEM((LBS + 16,), jnp.int32),
            pltpu.VMEM((2, GCH, 128), jnp.float32),
            pltpu.VMEM((BPW + 16, 128), jnp.float32),
            pltpu.VMEM((1, 16), jnp.int32),
            pltpu.SemaphoreType.DMA,
            pltpu.SemaphoreType.DMA,
        ],
        compiler_params=_cp,
    )
    def _agg(h_hbm, lsrc_hbm, ldst_hbm, cnt_hbm, out_hbm,
             bsrc, bdst, rows2, acc, cvec, sem0, sem1):
        c = lax.axis_index("c")
        s = lax.axis_index("s")
        wid = c * NS + s
        nbase = wid * BPW
        sems = (sem0, sem1)

        ival = jnp.full((16,), init, jnp.float32)

        @pl.loop(0, BPW + 16)
        def _(i):
            for j in range(8):
                acc[i, pl.ds(j * 16, 16)] = ival

        pltpu.sync_copy(cnt_hbm.at[pl.ds(wid * 16, 16)], cvec.at[0])
        n = lax.reduce_max(cvec[0, :], (0,))
        nblk = lax.shift_right_logical(n + LBS - 1, 11)

        def gissue(gb, ck):
            pltpu.async_copy(
                h_hbm.at[bsrc.at[pl.ds(ck * GCH, GCH)]], rows2.at[gb], sems[gb])

        def gwait(gb, ck):
            pltpu.make_async_copy(
                h_hbm.at[bsrc.at[pl.ds(ck * GCH, GCH)]], rows2.at[gb],
                sems[gb]).wait()

        def compute(gb, ck):
            @pl.loop(0, GCH)
            def _(e):
                dl = bdst[pl.ds(ck * GCH + e, 16)][0] - nbase
                for j in range(8):
                    sl = pl.ds(j * 16, 16)
                    if op == "add":
                        acc[dl, sl] = acc[dl, sl] + rows2[gb, e, sl]
                    else:
                        acc[dl, sl] = jnp.maximum(acc[dl, sl], rows2[gb, e, sl])

        def block(b, carry):
            off = b * LBS
            pltpu.sync_copy(lsrc_hbm.at[wid].at[pl.ds(off, LBS)], bsrc)
            pltpu.sync_copy(ldst_hbm.at[wid].at[pl.ds(off, LBS)],
                            bdst.at[pl.ds(0, LBS)])
            rem = n - off
            pr = jnp.minimum(jnp.int32(LBS // PAIR),
                             lax.shift_right_logical(rem, 8))
            gissue(0, 0)

            def pair(p, cc):
                gissue(1, 2 * p + 1)
                gwait(0, 2 * p)
                compute(0, 2 * p)

                @pl.when(p + 1 < pr)
                def _():
                    gissue(0, 2 * p + 2)

                gwait(1, 2 * p + 1)
                compute(1, 2 * p + 1)
                return cc

            return lax.fori_loop(0, pr, pair, carry)

        lax.fori_loop(0, nblk, block, jnp.int32(0))

        @pl.loop(0, BPW // 64)
        def _(k):
            r = k * 64
            pltpu.sync_copy(acc.at[pl.ds(r, 64)],
                            out_hbm.at[pl.ds(nbase + r, 64)])

    return _agg


_agg_max = _make_agg("max")


def _mean_agg(h, lsrc, ldst, cnts, deg):
    sums = _sum_sc(h, lsrc, ldst, cnts)
    mean = sums[:N] / jnp.maximum(deg, 1.0)[:, None]
    return jnp.where((deg > 0)[:, None], mean, h)


def _max_agg(h, lsrc, ldst, cnts, deg):
    mx = _agg_max(h, lsrc, ldst, cnts)
    return jnp.where((deg > 0)[:, None], mx[:N], h)


def _dense_bn_body(h_ref, w_ref, b_ref, g_ref, bb_ref, o_ref):
    h = h_ref[...]
    a = jnp.maximum(
        jnp.dot(h, w_ref[...], preferred_element_type=jnp.float32) + b_ref[...],
        0.0,
    )
    mu = jnp.mean(a, axis=0, keepdims=True)
    var = jnp.mean((a - mu) ** 2, axis=0, keepdims=True)
    o_ref[...] = (a - mu) * lax.rsqrt(var + 1e-5) * g_ref[...] + bb_ref[...]


def _dense_bn(h, W, b, g, bb):
    return pl.pallas_call(
        _dense_bn_body,
        out_shape=jax.ShapeDtypeStruct((N, H), jnp.float32),
    )(h, W, b.reshape(1, H), g.reshape(1, H), bb.reshape(1, H))


def _head_body(h_ref, wd_ref, bd_ref, gf_ref, bf_ref, wp_ref, bp_ref, o_ref):
    h = h_ref[...]
    a = jnp.maximum(
        jnp.dot(h, wd_ref[...], preferred_element_type=jnp.float32) + bd_ref[...],
        0.0,
    )
    mu = jnp.mean(a, axis=0, keepdims=True)
    var = jnp.mean((a - mu) ** 2, axis=0, keepdims=True)
    hb = (a - mu) * lax.rsqrt(var + 1e-5) * gf_ref[...] + bf_ref[...]
    hg = jnp.tanh(jnp.mean(hb, axis=0, keepdims=True))
    o_ref[...] = jnp.dot(hg, wp_ref[...], preferred_element_type=jnp.float32) + bp_ref[...]


def kernel(x, edge_index, W1, b1, g1, bb1, W2, b2, g2, bb2, Wd, bd, gf, bf, Wp, bp):
    src = edge_index[0]
    dst = edge_index[1]

    deg = jax.ops.segment_sum(jnp.ones((E,), jnp.float32), dst, num_segments=N)
    lsrc, ldst, cnts = _bin_edges_sc(src, dst)
    h = _mean_agg(x, lsrc, ldst, cnts, deg)
    h = _dense_bn(h, W1, b1, g1, bb1)
    h = _max_agg(h, lsrc, ldst, cnts, deg)
    h = _mean_agg(h, lsrc, ldst, cnts, deg)
    h = _dense_bn(h, W2, b2, g2, bb2)
    h = _max_agg(h, lsrc, ldst, cnts, deg)

    out = pl.pallas_call(
        _head_body,
        out_shape=jax.ShapeDtypeStruct((1, 1), jnp.float32),
    )(h, Wd, bd.reshape(1, H), gf.reshape(1, H), bf.reshape(1, H), Wp, bp.reshape(1, 1))
    return out


# speedup vs baseline: 2.1253x; 1.1549x over previous
import dataclasses
import functools

import jax
import jax.numpy as jnp
from jax import lax
from jax.experimental import pallas as pl
from jax.experimental.pallas import tpu as pltpu
from jax.experimental.pallas import tpu_sc as plsc

N = 10000
E = 320000
D = 128
H = 128

NPAD = 10240
NC = 2
NS = 16
NW = NC * NS
BPW = NPAD // NW

SCAN = 4000
NSTEP = E // SCAN
CHV = 2048
CAPT = (E // CHV + 2) * CHV
GCH = 128
PAIR = 2 * GCH
LBS = 2048

_mesh = plsc.VectorSubcoreMesh(core_axis_name="c", subcore_axis_name="s")
_cp = pltpu.CompilerParams()
if "needs_layout_passes" in pltpu.CompilerParams.__dataclass_fields__:
    _cp = dataclasses.replace(_cp, needs_layout_passes=False)


@functools.partial(
    pl.kernel,
    mesh=_mesh,
    out_type=[
        jax.ShapeDtypeStruct((NW, CAPT), jnp.int32),
        jax.ShapeDtypeStruct((NW, CAPT), jnp.int32),
        jax.ShapeDtypeStruct((NW * 16,), jnp.int32),
    ],
    scratch_types=[
        pltpu.VMEM((SCAN,), jnp.int32),
        pltpu.VMEM((SCAN,), jnp.int32),
        pltpu.VMEM((SCAN,), jnp.int32),
        pltpu.VMEM((SCAN,), jnp.int32),
        pltpu.VMEM((CHV + 16,), jnp.int32),
        pltpu.VMEM((CHV + 16,), jnp.int32),
        pltpu.VMEM((16,), jnp.int32),
        pltpu.SemaphoreType.DMA,
        pltpu.SemaphoreType.DMA,
    ],
    compiler_params=_cp,
)
def _bin_edges_sc(src_hbm, dst_hbm, lsrc_hbm, ldst_hbm, cnt_hbm,
                  in_src0, in_src1, in_dst0, in_dst1,
                  buf_src, buf_dst, cstage, sem0, sem1):
    c = lax.axis_index("c")
    s = lax.axis_index("s")
    wid = c * NS + s
    lo = wid * BPW
    hi = lo + BPW
    src_sent = wid * 311 % N
    dst_sent = lo + BPW
    sems = (sem0, sem1)
    srcb = (in_src0, in_src1)
    dstb = (in_dst0, in_dst1)

    def issue(b, i):
        off = i * SCAN
        pltpu.async_copy(src_hbm.at[pl.ds(off, SCAN)], srcb[b], sems[b])
        pltpu.async_copy(dst_hbm.at[pl.ds(off, SCAN)], dstb[b], sems[b])

    def wait(b, i):
        off = i * SCAN
        pltpu.make_async_copy(src_hbm.at[pl.ds(off, SCAN)], srcb[b], sems[b]).wait()
        pltpu.make_async_copy(dst_hbm.at[pl.ds(off, SCAN)], dstb[b], sems[b]).wait()

    def flush(carry):
        cnt, nf = carry
        pltpu.sync_copy(buf_src.at[pl.ds(0, CHV)],
                        lsrc_hbm.at[wid].at[pl.ds(nf * CHV, CHV)])
        pltpu.sync_copy(buf_dst.at[pl.ds(0, CHV)],
                        ldst_hbm.at[wid].at[pl.ds(nf * CHV, CHV)])
        ts = buf_src[pl.ds(CHV, 16)]
        td = buf_dst[pl.ds(CHV, 16)]
        buf_src[pl.ds(0, 16)] = ts
        buf_dst[pl.ds(0, 16)] = td
        return (cnt - CHV, nf + 1)

    def process(b, carry):
        def sub(kk, cc):
            cnt, nf = cc
            dv = dstb[b][pl.ds(kk * 16, 16)]
            sv = srcb[b][pl.ds(kk * 16, 16)]
            m = (dv >= lo) & (dv < hi)
            plsc.store_compressed(buf_src.at[pl.ds(cnt, 16)], sv, mask=m)
            plsc.store_compressed(buf_dst.at[pl.ds(cnt, 16)], dv, mask=m)
            cnt = cnt + plsc.all_reduce_population_count(m)[0]
            return lax.cond(cnt >= CHV, flush, lambda cc2: cc2, (cnt, nf))

        return lax.fori_loop(0, SCAN // 16, sub, carry)

    issue(0, 0)

    def pair_step(p, carry):
        issue(1, 2 * p + 1)
        wait(0, 2 * p)
        carry = process(0, carry)

        @pl.when(2 * p + 2 < NSTEP)
        def _():
            issue(0, 2 * p + 2)

        wait(1, 2 * p + 1)
        return process(1, carry)

    cnt, nf = lax.fori_loop(0, NSTEP // 2, pair_step,
                            (jnp.int32(0), jnp.int32(0)))

    sent_s = jnp.full((16,), src_sent, jnp.int32)
    sent_d = jnp.full((16,), dst_sent, jnp.int32)
    buf_src[pl.ds(cnt, 16)] = sent_s
    buf_dst[pl.ds(cnt, 16)] = sent_d
    cnt = cnt + ((16 - (cnt & 15)) & 15)

    def pad16(j, cnt2):
        @pl.when((cnt2 & (PAIR - 1)) != 0)
        def _():
            buf_src[pl.ds(cnt2, 16)] = sent_s
            buf_dst[pl.ds(cnt2, 16)] = sent_d

        return lax.cond((cnt2 & (PAIR - 1)) != 0,
                        lambda v: v + 16, lambda v: v, cnt2)

    cnt = lax.fori_loop(0, PAIR // 16 - 1, pad16, cnt)

    pltpu.sync_copy(buf_src.at[pl.ds(0, CHV)],
                    lsrc_hbm.at[wid].at[pl.ds(nf * CHV, CHV)])
    pltpu.sync_copy(buf_dst.at[pl.ds(0, CHV)],
                    ldst_hbm.at[wid].at[pl.ds(nf * CHV, CHV)])

    total = nf * CHV + cnt
    cstage[pl.ds(0, 16)] = jnp.full((16,), total, jnp.int32)
    pltpu.sync_copy(cstage.at[pl.ds(0, 16)], cnt_hbm.at[pl.ds(wid * 16, 16)])


NHALF = NPAD // 2


@functools.partial(
    pl.kernel,
    mesh=_mesh,
    out_type=jax.ShapeDtypeStruct((NPAD, 128), jnp.float32),
    scratch_types=[
        pltpu.VMEM((LBS,), jnp.int32),
        pltpu.VMEM((LBS,), jnp.int32),
        pltpu.VMEM((2, GCH, 128), jnp.float32),
        pltpu.VMEM((2, 128), jnp.int32),
        pltpu.VMEM((107, 128), jnp.float32),
        pltpu.VMEM((1, 16), jnp.int32),
        pltpu.VMEM_SHARED((NHALF + 16, 128), jnp.float32),
        pltpu.SemaphoreType.DMA,
        pltpu.SemaphoreType.DMA,
    ],
    compiler_params=_cp,
)
def _sum_sc(h_hbm, lsrc_hbm, ldst_hbm, cnt_hbm, out_hbm,
            bsrc, bdst, rows2, ilocb, zbuf, cvec, acc_sh, sem0, sem1):
    c = lax.axis_index("c")
    s = lax.axis_index("s")
    wid = c * NS + s
    sc_base = c * NHALF
    sent_val = (wid + 1) * BPW
    sems = (sem0, sem1)

    zero16 = jnp.zeros((16,), jnp.float32)

    @pl.loop(0, 107)
    def _(i):
        for j in range(8):
            zbuf[i, pl.ds(j * 16, 16)] = zero16

    @pl.loop(0, 3)
    def _(k):
        pltpu.sync_copy(zbuf.at[pl.ds(0, 107)],
                        acc_sh.at[pl.ds(s * 321 + k * 107, 107)])

    plsc.subcore_barrier()

    pltpu.sync_copy(cnt_hbm.at[pl.ds(wid * 16, 16)], cvec.at[0])
    n = lax.reduce_max(cvec[0, :], (0,))
    nblk = lax.shift_right_logical(n + LBS - 1, 11)

    def gissue(gb, ck):
        pltpu.async_copy(
            h_hbm.at[bsrc.at[pl.ds(ck * GCH, GCH)]], rows2.at[gb], sems[gb])

    def gwait(gb, ck):
        pltpu.make_async_copy(
            h_hbm.at[bsrc.at[pl.ds(ck * GCH, GCH)]], rows2.at[gb],
            sems[gb]).wait()

    def scatter(gb, ck):
        for j in range(8):
            dv = bdst[pl.ds(ck * GCH + j * 16, 16)]
            loc = jnp.where(dv == sent_val, jnp.int32(NHALF), dv - sc_base)
            ilocb[gb, pl.ds(j * 16, 16)] = loc
        pltpu.sync_copy(rows2.at[gb], acc_sh.at[ilocb.at[gb]], add=True)

    def block(b, carry):
        off = b * LBS
        pltpu.sync_copy(lsrc_hbm.at[wid].at[pl.ds(off, LBS)], bsrc)
        pltpu.sync_copy(ldst_hbm.at[wid].at[pl.ds(off, LBS)], bdst)
        rem = n - off
        pr = jnp.minimum(jnp.int32(LBS // PAIR),
                         lax.shift_right_logical(rem, 8))
        gissue(0, 0)

        def pair(p, cc):
            gissue(1, 2 * p + 1)
            gwait(0, 2 * p)
            scatter(0, 2 * p)

            @pl.when(p + 1 < pr)
            def _():
                gissue(0, 2 * p + 2)

            gwait(1, 2 * p + 1)
            scatter(1, 2 * p + 1)
            return cc

        return lax.fori_loop(0, pr, pair, carry)

    lax.fori_loop(0, nblk, block, jnp.int32(0))

    @pl.loop(0, BPW // 64)
    def _(k):
        r = s * BPW + k * 64
        pltpu.sync_copy(acc_sh.at[pl.ds(r, 64)],
                        out_hbm.at[pl.ds(sc_base + r, 64)])


def _make_agg(op):
    del op

    @functools.partial(
        pl.kernel,
        mesh=_mesh,
        out_type=jax.ShapeDtypeStruct((NPAD, 128), jnp.float32),
        scratch_types=[
            pltpu.VMEM((LBS,), jnp.int32),
            pltpu.VMEM((LBS + 16,), jnp.int32),
            pltpu.VMEM((2, GCH, 128), jnp.float32),
            pltpu.VMEM((BPW + 16, 128), jnp.float32),
            pltpu.VMEM((1, 16), jnp.int32),
            pltpu.SemaphoreType.DMA,
            pltpu.SemaphoreType.DMA,
        ],
        compiler_params=_cp,
    )
    def _agg(h_hbm, lsrc_hbm, ldst_hbm, cnt_hbm, out_hbm,
             bsrc, bdst, rows2, acc, cvec, sem0, sem1):
        c = lax.axis_index("c")
        s = lax.axis_index("s")
        wid = c * NS + s
        nbase = wid * BPW
        sems = (sem0, sem1)

        ival = jnp.full((16,), -3.4e38, jnp.float32)

        @pl.loop(0, BPW + 16)
        def _(i):
            for j in range(8):
                acc[i, pl.ds(j * 16, 16)] = ival

        pltpu.sync_copy(cnt_hbm.at[pl.ds(wid * 16, 16)], cvec.at[0])
        n = lax.reduce_max(cvec[0, :], (0,))
        nblk = lax.shift_right_logical(n + LBS - 1, 11)

        def gissue(gb, ck):
            pltpu.async_copy(
                h_hbm.at[bsrc.at[pl.ds(ck * GCH, GCH)]], rows2.at[gb], sems[gb])

        def gwait(gb, ck):
            pltpu.make_async_copy(
                h_hbm.at[bsrc.at[pl.ds(ck * GCH, GCH)]], rows2.at[gb],
                sems[gb]).wait()

        def compute(gb, ck):
            @pl.loop(0, GCH // 16)
            def _(g):
                dvec = bdst[pl.ds(ck * GCH + g * 16, 16)] - nbase
                for j in range(16):
                    dl = dvec[j]
                    for f in range(8):
                        sl = pl.ds(f * 16, 16)
                        acc[dl, sl] = jnp.maximum(acc[dl, sl],
                                                  rows2[gb, g * 16 + j, sl])

        def block(b, carry):
            off = b * LBS
            pltpu.sync_copy(lsrc_hbm.at[wid].at[pl.ds(off, LBS)], bsrc)
            pltpu.sync_copy(ldst_hbm.at[wid].at[pl.ds(off, LBS)],
                            bdst.at[pl.ds(0, LBS)])
            rem = n - off
            pr = jnp.minimum(jnp.int32(LBS // PAIR),
                             lax.shift_right_logical(rem, 8))
            gissue(0, 0)

            def pair(p, cc):
                gissue(1, 2 * p + 1)
                gwait(0, 2 * p)
                compute(0, 2 * p)

                @pl.when(p + 1 < pr)
                def _():
                    gissue(0, 2 * p + 2)

                gwait(1, 2 * p + 1)
                compute(1, 2 * p + 1)
                return cc

            return lax.fori_loop(0, pr, pair, carry)

        lax.fori_loop(0, nblk, block, jnp.int32(0))

        @pl.loop(0, BPW // 64)
        def _(k):
            r = k * 64
            pltpu.sync_copy(acc.at[pl.ds(r, 64)],
                            out_hbm.at[pl.ds(nbase + r, 64)])

    return _agg


_agg_max = _make_agg("max")


def _mean_agg(h, lsrc, ldst, cnts, deg):
    sums = _sum_sc(h, lsrc, ldst, cnts)
    mean = sums[:N] / jnp.maximum(deg, 1.0)[:, None]
    return jnp.where((deg > 0)[:, None], mean, h)


def _max_agg(h, lsrc, ldst, cnts, deg):
    mx = _agg_max(h, lsrc, ldst, cnts)
    return jnp.where((deg > 0)[:, None], mx[:N], h)


def _dense_bn_body(h_ref, w_ref, b_ref, g_ref, bb_ref, o_ref):
    h = h_ref[...]
    a = jnp.maximum(
        jnp.dot(h, w_ref[...], preferred_element_type=jnp.float32) + b_ref[...],
        0.0,
    )
    mu = jnp.mean(a, axis=0, keepdims=True)
    var = jnp.mean((a - mu) ** 2, axis=0, keepdims=True)
    o_ref[...] = (a - mu) * lax.rsqrt(var + 1e-5) * g_ref[...] + bb_ref[...]


def _dense_bn(h, W, b, g, bb):
    return pl.pallas_call(
        _dense_bn_body,
        out_shape=jax.ShapeDtypeStruct((N, H), jnp.float32),
    )(h, W, b.reshape(1, H), g.reshape(1, H), bb.reshape(1, H))


def _head_body(h_ref, wd_ref, bd_ref, gf_ref, bf_ref, wp_ref, bp_ref, o_ref):
    h = h_ref[...]
    a = jnp.maximum(
        jnp.dot(h, wd_ref[...], preferred_element_type=jnp.float32) + bd_ref[...],
        0.0,
    )
    mu = jnp.mean(a, axis=0, keepdims=True)
    var = jnp.mean((a - mu) ** 2, axis=0, keepdims=True)
    hb = (a - mu) * lax.rsqrt(var + 1e-5) * gf_ref[...] + bf_ref[...]
    hg = jnp.tanh(jnp.mean(hb, axis=0, keepdims=True))
    o_ref[...] = jnp.dot(hg, wp_ref[...], preferred_element_type=jnp.float32) + bp_ref[...]


def kernel(x, edge_index, W1, b1, g1, bb1, W2, b2, g2, bb2, Wd, bd, gf, bf, Wp, bp):
    src = edge_index[0]
    dst = edge_index[1]

    deg = jax.ops.segment_sum(jnp.ones((E,), jnp.float32), dst, num_segments=N)
    lsrc, ldst, cnts = _bin_edges_sc(src, dst)
    h = _mean_agg(x, lsrc, ldst, cnts, deg)
    h = _dense_bn(h, W1, b1, g1, bb1)
    h = _max_agg(h, lsrc, ldst, cnts, deg)
    h = _mean_agg(h, lsrc, ldst, cnts, deg)
    h = _dense_bn(h, W2, b2, g2, bb2)
    h = _max_agg(h, lsrc, ldst, cnts, deg)

    out = pl.pallas_call(
        _head_body,
        out_shape=jax.ShapeDtypeStruct((1, 1), jnp.float32),
    )(h, Wd, bd.reshape(1, H), gf.reshape(1, H), bf.reshape(1, H), Wp, bp.reshape(1, 1))
    return out


# speedup vs baseline: 2.4858x; 1.1696x over previous
import dataclasses
import functools

import jax
import jax.numpy as jnp
from jax import lax
from jax.experimental import pallas as pl
from jax.experimental.pallas import tpu as pltpu
from jax.experimental.pallas import tpu_sc as plsc

N = 10000
E = 320000
D = 128
H = 128

NPAD = 10240
NC = 2
NS = 16
NW = NC * NS
BPW = NPAD // NW

SCAN = 6400
NSTEP = E // SCAN
CHV = 2048
CAPT = (E // CHV + 2) * CHV
GCH = 128
PAIR = 2 * GCH
LBS = 2048

_mesh = plsc.VectorSubcoreMesh(core_axis_name="c", subcore_axis_name="s")
_cp = pltpu.CompilerParams()
if "needs_layout_passes" in pltpu.CompilerParams.__dataclass_fields__:
    _cp = dataclasses.replace(_cp, needs_layout_passes=False)


@functools.partial(
    pl.kernel,
    mesh=_mesh,
    out_type=[
        jax.ShapeDtypeStruct((NW, CAPT), jnp.int32),
        jax.ShapeDtypeStruct((NW, CAPT), jnp.int32),
        jax.ShapeDtypeStruct((NW * 16,), jnp.int32),
    ],
    scratch_types=[
        pltpu.VMEM((SCAN,), jnp.int32),
        pltpu.VMEM((SCAN,), jnp.int32),
        pltpu.VMEM((SCAN,), jnp.int32),
        pltpu.VMEM((SCAN,), jnp.int32),
        pltpu.VMEM((CHV + 272,), jnp.int32),
        pltpu.VMEM((CHV + 272,), jnp.int32),
        pltpu.VMEM((16,), jnp.int32),
        pltpu.SemaphoreType.DMA,
        pltpu.SemaphoreType.DMA,
    ],
    compiler_params=_cp,
)
def _bin_edges_sc(src_hbm, dst_hbm, lsrc_hbm, ldst_hbm, cnt_hbm,
                  in_src0, in_src1, in_dst0, in_dst1,
                  buf_src, buf_dst, cstage, sem0, sem1):
    c = lax.axis_index("c")
    s = lax.axis_index("s")
    wid = c * NS + s
    lo = wid * BPW
    hi = lo + BPW
    src_sent = wid * 311 % N
    dst_sent = lo + BPW
    sems = (sem0, sem1)
    srcb = (in_src0, in_src1)
    dstb = (in_dst0, in_dst1)

    def issue(b, i):
        off = i * SCAN
        pltpu.async_copy(src_hbm.at[pl.ds(off, SCAN)], srcb[b], sems[b])
        pltpu.async_copy(dst_hbm.at[pl.ds(off, SCAN)], dstb[b], sems[b])

    def wait(b, i):
        off = i * SCAN
        pltpu.make_async_copy(src_hbm.at[pl.ds(off, SCAN)], srcb[b], sems[b]).wait()
        pltpu.make_async_copy(dst_hbm.at[pl.ds(off, SCAN)], dstb[b], sems[b]).wait()

    def flush(carry):
        cnt, nf = carry
        pltpu.sync_copy(buf_src.at[pl.ds(0, CHV)],
                        lsrc_hbm.at[wid].at[pl.ds(nf * CHV, CHV)])
        pltpu.sync_copy(buf_dst.at[pl.ds(0, CHV)],
                        ldst_hbm.at[wid].at[pl.ds(nf * CHV, CHV)])
        for j in range(16):
            ts = buf_src[pl.ds(CHV + j * 16, 16)]
            td = buf_dst[pl.ds(CHV + j * 16, 16)]
            buf_src[pl.ds(j * 16, 16)] = ts
            buf_dst[pl.ds(j * 16, 16)] = td
        return (cnt - CHV, nf + 1)

    def maybe_flush(cc):
        return lax.cond(cc[0] >= CHV, flush, lambda cc2: cc2, cc)

    def process(b, carry):
        def group(g, cc):
            cnt, nf = cc
            for kk in range(16):
                dv = dstb[b][pl.ds(g * 256 + kk * 16, 16)]
                sv = srcb[b][pl.ds(g * 256 + kk * 16, 16)]
                m = (dv >= lo) & (dv < hi)
                plsc.store_compressed(buf_src.at[pl.ds(cnt, 16)], sv, mask=m)
                plsc.store_compressed(buf_dst.at[pl.ds(cnt, 16)], dv, mask=m)
                cnt = cnt + plsc.all_reduce_population_count(m)[0]
            return maybe_flush((cnt, nf))

        return lax.fori_loop(0, SCAN // 256, group, carry)

    issue(0, 0)

    def pair_step(p, carry):
        issue(1, 2 * p + 1)
        wait(0, 2 * p)
        carry = process(0, carry)

        @pl.when(2 * p + 2 < NSTEP)
        def _():
            issue(0, 2 * p + 2)

        wait(1, 2 * p + 1)
        return process(1, carry)

    cnt, nf = lax.fori_loop(0, NSTEP // 2, pair_step,
                            (jnp.int32(0), jnp.int32(0)))

    sent_s = jnp.full((16,), src_sent, jnp.int32)
    sent_d = jnp.full((16,), dst_sent, jnp.int32)
    buf_src[pl.ds(cnt, 16)] = sent_s
    buf_dst[pl.ds(cnt, 16)] = sent_d
    cnt = cnt + ((16 - (cnt & 15)) & 15)

    def pad16(j, cnt2):
        @pl.when((cnt2 & (PAIR - 1)) != 0)
        def _():
            buf_src[pl.ds(cnt2, 16)] = sent_s
            buf_dst[pl.ds(cnt2, 16)] = sent_d

        return lax.cond((cnt2 & (PAIR - 1)) != 0,
                        lambda v: v + 16, lambda v: v, cnt2)

    cnt = lax.fori_loop(0, PAIR // 16 - 1, pad16, cnt)
    cnt, nf = maybe_flush((cnt, nf))

    pltpu.sync_copy(buf_src.at[pl.ds(0, CHV)],
                    lsrc_hbm.at[wid].at[pl.ds(nf * CHV, CHV)])
    pltpu.sync_copy(buf_dst.at[pl.ds(0, CHV)],
                    ldst_hbm.at[wid].at[pl.ds(nf * CHV, CHV)])

    total = nf * CHV + cnt
    cstage[pl.ds(0, 16)] = jnp.full((16,), total, jnp.int32)
    pltpu.sync_copy(cstage.at[pl.ds(0, 16)], cnt_hbm.at[pl.ds(wid * 16, 16)])


NHALF = NPAD // 2


@functools.partial(
    pl.kernel,
    mesh=_mesh,
    out_type=jax.ShapeDtypeStruct((NPAD, 128), jnp.float32),
    scratch_types=[
        pltpu.VMEM((LBS,), jnp.int32),
        pltpu.VMEM((LBS,), jnp.int32),
        pltpu.VMEM((2, GCH, 128), jnp.float32),
        pltpu.VMEM((2, 128), jnp.int32),
        pltpu.VMEM((107, 128), jnp.float32),
        pltpu.VMEM((1, 16), jnp.int32),
        pltpu.VMEM_SHARED((NHALF + 16, 128), jnp.float32),
        pltpu.SemaphoreType.DMA,
        pltpu.SemaphoreType.DMA,
    ],
    compiler_params=_cp,
)
def _sum_sc(h_hbm, lsrc_hbm, ldst_hbm, cnt_hbm, out_hbm,
            bsrc, bdst, rows2, ilocb, zbuf, cvec, acc_sh, sem0, sem1):
    c = lax.axis_index("c")
    s = lax.axis_index("s")
    wid = c * NS + s
    sc_base = c * NHALF
    sent_val = (wid + 1) * BPW
    sems = (sem0, sem1)

    zero16 = jnp.zeros((16,), jnp.float32)

    @pl.loop(0, 107)
    def _(i):
        for j in range(8):
            zbuf[i, pl.ds(j * 16, 16)] = zero16

    @pl.loop(0, 3)
    def _(k):
        pltpu.sync_copy(zbuf.at[pl.ds(0, 107)],
                        acc_sh.at[pl.ds(s * 321 + k * 107, 107)])

    plsc.subcore_barrier()

    pltpu.sync_copy(cnt_hbm.at[pl.ds(wid * 16, 16)], cvec.at[0])
    n = lax.reduce_max(cvec[0, :], (0,))
    nblk = lax.shift_right_logical(n + LBS - 1, 11)

    def gissue(gb, ck):
        pltpu.async_copy(
            h_hbm.at[bsrc.at[pl.ds(ck * GCH, GCH)]], rows2.at[gb], sems[gb])

    def gwait(gb, ck):
        pltpu.make_async_copy(
            h_hbm.at[bsrc.at[pl.ds(ck * GCH, GCH)]], rows2.at[gb],
            sems[gb]).wait()

    def scatter(gb, ck):
        for j in range(8):
            dv = bdst[pl.ds(ck * GCH + j * 16, 16)]
            loc = jnp.where(dv == sent_val, jnp.int32(NHALF), dv - sc_base)
            ilocb[gb, pl.ds(j * 16, 16)] = loc
        pltpu.sync_copy(rows2.at[gb], acc_sh.at[ilocb.at[gb]], add=True)

    def block(b, carry):
        off = b * LBS
        pltpu.sync_copy(lsrc_hbm.at[wid].at[pl.ds(off, LBS)], bsrc)
        pltpu.sync_copy(ldst_hbm.at[wid].at[pl.ds(off, LBS)], bdst)
        rem = n - off
        pr = jnp.minimum(jnp.int32(LBS // PAIR),
                         lax.shift_right_logical(rem, 8))
        gissue(0, 0)

        def pair(p, cc):
            gissue(1, 2 * p + 1)
            gwait(0, 2 * p)
            scatter(0, 2 * p)

            @pl.when(p + 1 < pr)
            def _():
                gissue(0, 2 * p + 2)

            gwait(1, 2 * p + 1)
            scatter(1, 2 * p + 1)
            return cc

        return lax.fori_loop(0, pr, pair, carry)

    lax.fori_loop(0, nblk, block, jnp.int32(0))

    @pl.loop(0, BPW // 64)
    def _(k):
        r = s * BPW + k * 64
        pltpu.sync_copy(acc_sh.at[pl.ds(r, 64)],
                        out_hbm.at[pl.ds(sc_base + r, 64)])


def _make_agg(op):
    del op

    @functools.partial(
        pl.kernel,
        mesh=_mesh,
        out_type=jax.ShapeDtypeStruct((NPAD, 128), jnp.float32),
        scratch_types=[
            pltpu.VMEM((LBS,), jnp.int32),
            pltpu.VMEM((LBS + 16,), jnp.int32),
            pltpu.VMEM((2, GCH, 128), jnp.float32),
            pltpu.VMEM((BPW + 16, 128), jnp.float32),
            pltpu.VMEM((1, 16), jnp.int32),
            pltpu.SemaphoreType.DMA,
            pltpu.SemaphoreType.DMA,
        ],
        compiler_params=_cp,
    )
    def _agg(h_hbm, lsrc_hbm, ldst_hbm, cnt_hbm, out_hbm,
             bsrc, bdst, rows2, acc, cvec, sem0, sem1):
        c = lax.axis_index("c")
        s = lax.axis_index("s")
        wid = c * NS + s
        nbase = wid * BPW
        sems = (sem0, sem1)

        ival = jnp.full((16,), -3.4e38, jnp.float32)

        @pl.loop(0, BPW + 16)
        def _(i):
            for j in range(8):
                acc[i, pl.ds(j * 16, 16)] = ival

        pltpu.sync_copy(cnt_hbm.at[pl.ds(wid * 16, 16)], cvec.at[0])
        n = lax.reduce_max(cvec[0, :], (0,))
        nblk = lax.shift_right_logical(n + LBS - 1, 11)

        def gissue(gb, ck):
            pltpu.async_copy(
                h_hbm.at[bsrc.at[pl.ds(ck * GCH, GCH)]], rows2.at[gb], sems[gb])

        def gwait(gb, ck):
            pltpu.make_async_copy(
                h_hbm.at[bsrc.at[pl.ds(ck * GCH, GCH)]], rows2.at[gb],
                sems[gb]).wait()

        def compute(gb, ck):
            @pl.loop(0, GCH // 16)
            def _(g):
                dvec = bdst[pl.ds(ck * GCH + g * 16, 16)] - nbase
                for j in range(16):
                    dl = dvec[j]
                    for f in range(8):
                        sl = pl.ds(f * 16, 16)
                        acc[dl, sl] = jnp.maximum(acc[dl, sl],
                                                  rows2[gb, g * 16 + j, sl])

        def block(b, carry):
            off = b * LBS
            pltpu.sync_copy(lsrc_hbm.at[wid].at[pl.ds(off, LBS)], bsrc)
            pltpu.sync_copy(ldst_hbm.at[wid].at[pl.ds(off, LBS)],
                            bdst.at[pl.ds(0, LBS)])
            rem = n - off
            pr = jnp.minimum(jnp.int32(LBS // PAIR),
                             lax.shift_right_logical(rem, 8))
            gissue(0, 0)

            def pair(p, cc):
                gissue(1, 2 * p + 1)
                gwait(0, 2 * p)
                compute(0, 2 * p)

                @pl.when(p + 1 < pr)
                def _():
                    gissue(0, 2 * p + 2)

                gwait(1, 2 * p + 1)
                compute(1, 2 * p + 1)
                return cc

            return lax.fori_loop(0, pr, pair, carry)

        lax.fori_loop(0, nblk, block, jnp.int32(0))

        @pl.loop(0, BPW // 64)
        def _(k):
            r = k * 64
            pltpu.sync_copy(acc.at[pl.ds(r, 64)],
                            out_hbm.at[pl.ds(nbase + r, 64)])

    return _agg


_agg_max = _make_agg("max")


def _mean_agg(h, lsrc, ldst, cnts, deg):
    sums = _sum_sc(h, lsrc, ldst, cnts)
    mean = sums[:N] / jnp.maximum(deg, 1.0)[:, None]
    return jnp.where((deg > 0)[:, None], mean, h)


def _max_agg(h, lsrc, ldst, cnts, deg):
    mx = _agg_max(h, lsrc, ldst, cnts)
    return jnp.where((deg > 0)[:, None], mx[:N], h)


def _dense_bn_body(h_ref, w_ref, b_ref, g_ref, bb_ref, o_ref):
    h = h_ref[...]
    a = jnp.maximum(
        jnp.dot(h, w_ref[...], preferred_element_type=jnp.float32) + b_ref[...],
        0.0,
    )
    mu = jnp.mean(a, axis=0, keepdims=True)
    var = jnp.mean((a - mu) ** 2, axis=0, keepdims=True)
    o_ref[...] = (a - mu) * lax.rsqrt(var + 1e-5) * g_ref[...] + bb_ref[...]


def _dense_bn(h, W, b, g, bb):
    return pl.pallas_call(
        _dense_bn_body,
        out_shape=jax.ShapeDtypeStruct((N, H), jnp.float32),
    )(h, W, b.reshape(1, H), g.reshape(1, H), bb.reshape(1, H))


def _head_body(h_ref, wd_ref, bd_ref, gf_ref, bf_ref, wp_ref, bp_ref, o_ref):
    h = h_ref[...]
    a = jnp.maximum(
        jnp.dot(h, wd_ref[...], preferred_element_type=jnp.float32) + bd_ref[...],
        0.0,
    )
    mu = jnp.mean(a, axis=0, keepdims=True)
    var = jnp.mean((a - mu) ** 2, axis=0, keepdims=True)
    hb = (a - mu) * lax.rsqrt(var + 1e-5) * gf_ref[...] + bf_ref[...]
    hg = jnp.tanh(jnp.mean(hb, axis=0, keepdims=True))
    o_ref[...] = jnp.dot(hg, wp_ref[...], preferred_element_type=jnp.float32) + bp_ref[...]


def kernel(x, edge_index, W1, b1, g1, bb1, W2, b2, g2, bb2, Wd, bd, gf, bf, Wp, bp):
    src = edge_index[0]
    dst = edge_index[1]

    deg = jax.ops.segment_sum(jnp.ones((E,), jnp.float32), dst, num_segments=N)
    lsrc, ldst, cnts = _bin_edges_sc(src, dst)
    h = _mean_agg(x, lsrc, ldst, cnts, deg)
    h = _dense_bn(h, W1, b1, g1, bb1)
    h = _max_agg(h, lsrc, ldst, cnts, deg)
    h = _mean_agg(h, lsrc, ldst, cnts, deg)
    h = _dense_bn(h, W2, b2, g2, bb2)
    h = _max_agg(h, lsrc, ldst, cnts, deg)

    out = pl.pallas_call(
        _head_body,
        out_shape=jax.ShapeDtypeStruct((1, 1), jnp.float32),
    )(h, Wd, bd.reshape(1, H), gf.reshape(1, H), bf.reshape(1, H), Wp, bp.reshape(1, 1))
    return out
